# Initial kernel scaffold; baseline (speedup 1.0000x reference)
import jax
import jax.numpy as jnp
from jax.experimental import pallas as pl


def kernel(inputs, emb_weight):
    raise NotImplementedError("write your pallas kernel here")


# speedup vs baseline: 2.1507x; 2.1507x over previous
import jax
import jax.numpy as jnp
from jax import lax
from jax.experimental import pallas as pl
from jax.experimental.pallas import tpu as pltpu

_K = 1024
_D = 256
_N = 8192
_BM = 1024
_NB = _N // _BM
_E_WEIGHT = 0.25
_MANAGE_WEIGHT = 0.1


def _vq_body(x_ref, emb_ref, q_ref, stats_ref, sump_ref, acc_ref):
    i = pl.program_id(0)

    @pl.when(i == 0)
    def _init():
        sump_ref[...] = jnp.zeros_like(sump_ref)
        acc_ref[0] = 0.0
        acc_ref[1] = 0.0

    x = x_ref[...]
    emb = emb_ref[...]
    norm = jnp.sqrt(jnp.sum(x * x, axis=1, keepdims=True))
    xn = x / jnp.maximum(norm, 1e-12)
    s = jnp.sum(xn * xn, axis=1, keepdims=True)
    e2 = jnp.sum(emb * emb, axis=1)
    dots = lax.dot_general(xn, emb, (((1,), (1,)), ((), ())))
    d = (s + e2[None, :]) - 2.0 * dots

    m = jnp.max(d, axis=1, keepdims=True)
    ex = jnp.exp(d - m)
    se = jnp.sum(ex, axis=1, keepdims=True)
    p = ex / se
    ent = -p * jnp.log(p + 1e-8)

    mind = jnp.min(d, axis=1, keepdims=True)
    kiota = lax.broadcasted_iota(jnp.int32, d.shape, 1)
    first = jnp.min(jnp.where(d == mind, kiota, _K), axis=1, keepdims=True)
    onehot = (kiota == first).astype(jnp.float32)
    q = lax.dot_general(onehot, emb, (((1,), (0,)), ((), ())))
    q_ref[...] = q

    sump_ref[...] += jnp.sum(p, axis=0, keepdims=True)
    acc_ref[0] += jnp.sum(ent)
    acc_ref[1] += jnp.sum(mind)

    @pl.when(i == _NB - 1)
    def _fin():
        intra = acc_ref[0] / _N
        mse = acc_ref[1] / (_N * _D)
        avg_p = sump_ref[...] / _N
        inter = jnp.sum(avg_p * jnp.log(avg_p + 1e-8))
        lane = lax.broadcasted_iota(jnp.int32, (1, 128), 1)
        stats_ref[...] = jnp.where(
            lane == 0, intra,
            jnp.where(lane == 1, inter, jnp.where(lane == 2, mse, 0.0)))


def kernel(inputs, emb_weight):
    x = jnp.transpose(inputs, (0, 2, 3, 1)).reshape(_N, _D)
    q, stats = pl.pallas_call(
        _vq_body,
        grid=(_NB,),
        in_specs=[
            pl.BlockSpec((_BM, _D), lambda i: (i, 0)),
            pl.BlockSpec((_K, _D), lambda i: (0, 0)),
        ],
        out_specs=[
            pl.BlockSpec((_BM, _D), lambda i: (i, 0)),
            pl.BlockSpec((1, 128), lambda i: (0, 0)),
        ],
        out_shape=[
            jax.ShapeDtypeStruct((_N, _D), jnp.float32),
            jax.ShapeDtypeStruct((1, 128), jnp.float32),
        ],
        scratch_shapes=[
            pltpu.VMEM((1, _K), jnp.float32),
            pltpu.SMEM((2,), jnp.float32),
        ],
        compiler_params=pltpu.CompilerParams(
            dimension_semantics=("arbitrary",)),
    )(x, emb_weight)
    intra = stats[0, 0]
    inter = stats[0, 1]
    mse = stats[0, 2]
    loss = (mse + _E_WEIGHT * mse) + _MANAGE_WEIGHT * (intra + inter)
    out = jnp.transpose(q.reshape(8, 32, 32, _D), (0, 3, 1, 2))
    return (loss, out, mse, mse, intra, inter)
